# baseline (speedup 1.0000x reference)
import functools
import math

import jax
import jax.numpy as jnp
from jax import lax
from jax.experimental import pallas as pl
from jax.experimental.pallas import tpu as pltpu
from jax.experimental.pallas import tpu_sc as plsc

N = 10000
E = 320000
C = 128

NPAD = 10240
NROW = NPAD // 128
NCORE, NSUB = 2, 16
NWORK = NCORE * NSUB
BLK = 128
NBLK = 80
EPT = NBLK * BLK
EPAD = NWORK * EPT
WP = 80
WQ = 64
WB = 80

K1 = 5000
K2 = 2500
K3 = 1250

_HP = lax.Precision.HIGHEST


def _sc_agg_body(width, table_hbm, edges_hbm, zeros_hbm, out_hbm,
                 acc, src_all, dst_all, r0b, r1b, r2b, r3b,
                 g0, g1, g2, g3, s0, s1, s2, s3):
    cid = lax.axis_index("c")
    sid = lax.axis_index("s")
    wid = cid * NSUB + sid
    rpt = NPAD // NSUB
    row0 = sid * rpt
    rows = (r0b, r1b, r2b, r3b)
    gsem = (g0, g1, g2, g3)
    ssem = (s0, s1, s2, s3)
    pltpu.sync_copy(zeros_hbm.at[pl.ds(row0, rpt)], acc.at[pl.ds(row0, rpt)])
    pltpu.sync_copy(edges_hbm.at[wid], src_all)
    pltpu.sync_copy(edges_hbm.at[NWORK + wid], dst_all)
    plsc.subcore_barrier()

    for k in range(3):
        pltpu.async_copy(table_hbm.at[src_all.at[k]], rows[k], gsem[k])

    def phase(i, k):
        kp = (k + 3) % 4

        @pl.when(jnp.logical_and(i >= 1, i <= NBLK - 4))
        def _():
            pltpu.make_async_copy(rows[kp], acc.at[dst_all.at[i - 1]],
                                  ssem[kp]).wait()

        @pl.when(i <= NBLK - 4)
        def _():
            pltpu.async_copy(table_hbm.at[src_all.at[i + 3]], rows[kp],
                             gsem[kp])

        pltpu.make_async_copy(table_hbm.at[src_all.at[i]], rows[k],
                              gsem[k]).wait()
        pltpu.async_copy(rows[k], acc.at[dst_all.at[i]], ssem[k], add=True)

    def group(j, carry):
        for k in range(4):
            phase(4 * j + k, k)
        return carry

    lax.fori_loop(0, NBLK // 4, group, 0)
    for k in range(4):
        pltpu.make_async_copy(rows[k], acc.at[dst_all.at[NBLK - 4 + k]],
                              ssem[k]).wait()
    plsc.subcore_barrier()
    pltpu.sync_copy(acc.at[pl.ds(row0, rpt)],
                    out_hbm.at[pl.ds(cid * NPAD + row0, rpt)])


def _sc_aggregate(table, edges3, zeros, width):
    mesh = plsc.VectorSubcoreMesh(core_axis_name="c", subcore_axis_name="s")
    fn = pl.kernel(
        functools.partial(_sc_agg_body, width),
        out_type=jax.ShapeDtypeStruct((2 * NPAD, width), jnp.float32),
        mesh=mesh,
        scratch_types=[
            pltpu.VMEM_SHARED((NPAD, width), jnp.float32),
            pltpu.VMEM((NBLK, BLK), jnp.int32),
            pltpu.VMEM((NBLK, BLK), jnp.int32),
        ] + [pltpu.VMEM((BLK, width), jnp.float32)] * 4
          + [pltpu.SemaphoreType.DMA] * 8,
        compiler_params=pltpu.CompilerParams(use_tc_tiling_on_sc=False),
    )
    return fn(table, edges3, zeros)


def _select_topk(score80, live80, kk):
    bits = lax.bitcast_convert_type(score80, jnp.int32)
    key = bits ^ ((bits >> 31) & jnp.int32(0x7FFFFFFF))
    ku = lax.bitcast_convert_type(key, jnp.uint32) ^ jnp.uint32(0x80000000)
    ku = jnp.where(live80 > 0, ku, jnp.uint32(0))

    def rbody(t, prefix):
        cand = prefix | (jnp.uint32(0x80000000) >> t.astype(jnp.uint32))
        cnt = jnp.sum(jnp.where(ku >= cand, 1, 0))
        return jnp.where(cnt >= kk, cand, prefix)

    t = lax.fori_loop(0, 32, rbody, jnp.uint32(0), unroll=True)
    c1 = jnp.sum(jnp.where(ku > t, 1, 0))
    r = kk - c1
    idx = (lax.broadcasted_iota(jnp.int32, (NROW, 128), 0) * 128
           + lax.broadcasted_iota(jnp.int32, (NROW, 128), 1))
    eq = ku == t

    def bbody(_, lohi):
        lo, hi = lohi
        mid = (lo + hi) // 2
        cnt = jnp.sum(jnp.where(eq & (idx <= mid), 1, 0))
        good = cnt >= r
        return (jnp.where(good, lo, mid + 1), jnp.where(good, mid, hi))

    lo, _ = lax.fori_loop(0, 14, bbody,
                          (jnp.int32(0), jnp.int32(NPAD - 1)), unroll=True)
    sel = (ku > t) | (eq & (idx <= lo))
    return jnp.where((live80 > 0) & sel, 1.0, 0.0)


def _conv_out(feat_ref, dg_ref, h, Wl_ref, bl_ref, Wr_ref):
    agg = feat_ref[0:NPAD] + feat_ref[NPAD:2 * NPAD]
    deg = _grid_to_col(dg_ref[0:NROW] + dg_ref[NROW:2 * NROW])
    mean = agg / jnp.maximum(deg, 1.0)
    pre = (bl_ref[...] + mean
           + jnp.dot(h, Wr_ref[...], precision=_HP)) if Wl_ref is None else (
        jnp.dot(mean, Wl_ref[...], precision=_HP) + bl_ref[...]
        + jnp.dot(h, Wr_ref[...], precision=_HP))
    return jnp.maximum(pre, 0.0)


def _col_to_grid(col):
    return jnp.concatenate(
        [col[i * 128:(i + 1) * 128, :].T for i in range(NROW)], axis=0)


def _grid_to_col(g):
    return jnp.concatenate([g[i:i + 1, :].T for i in range(NROW)], axis=0)


def _pool(out, live80, p_ref, kk):
    p = p_ref[...]
    pnorm = jnp.sqrt(jnp.sum(p * p))
    sc_col = jnp.tanh(jnp.sum(out * p, axis=1, keepdims=True)
                      / (pnorm + 1e-16))
    score80 = _col_to_grid(sc_col)
    sel80 = _select_topk(score80, live80, kk)
    selc = _grid_to_col(sel80)
    hnew = out * (sc_col * selc)
    mx = jnp.max(jnp.where(selc > 0.0, hnew, -jnp.inf), axis=0, keepdims=True)
    mn = jnp.sum(hnew, axis=0, keepdims=True) * (1.0 / kk)
    return hnew, selc, sel80, jnp.concatenate([mx, mn], axis=1)


def _tc0_body(x_ref, g_ref, b_ref, h_ref):
    x = x_ref[...]
    mu = jnp.sum(x, axis=0, keepdims=True) * (1.0 / N)
    d = x - mu
    var = (jnp.sum(d * d, axis=0, keepdims=True)
           - (NPAD - N) * mu * mu) * (1.0 / N)
    h = d / jnp.sqrt(var + 1e-5) * g_ref[...] + b_ref[...]
    rmask = jnp.where(
        lax.broadcasted_iota(jnp.int32, (NPAD, 1), 0) < N, 1.0, 0.0)
    h_ref[...] = h * rmask


def _tc1_body(feat_ref, dg_ref, h0_ref, Wl_ref, bl_ref, Wr_ref, p_ref,
              hnew_ref, sel80_ref, xro_ref):
    out = _conv_out(feat_ref, dg_ref, h0_ref[...], Wl_ref, bl_ref, Wr_ref)
    idx = (lax.broadcasted_iota(jnp.int32, (NROW, 128), 0) * 128
           + lax.broadcasted_iota(jnp.int32, (NROW, 128), 1))
    live80 = jnp.where(idx < N, 1.0, 0.0)
    hnew, selc, sel80, xro = _pool(out, live80, p_ref, K1)
    hnew_ref[...] = hnew
    sel80_ref[...] = sel80
    xro_ref[...] = xro


def _tc2_body(feat_ref, dg_ref, h1_ref, live80_ref, Wl_ref, bl_ref, Wr_ref,
              p_ref, Wlnext_ref, y_ref, h2_ref, sel80_ref, xro_ref):
    out = _conv_out(feat_ref, dg_ref, h1_ref[...], Wl_ref, bl_ref, Wr_ref)
    hnew, selc, sel80, xro = _pool(out, live80_ref[...], p_ref, K2)
    y_ref[...] = jnp.dot(hnew, Wlnext_ref[...], precision=_HP)
    h2_ref[...] = hnew
    sel80_ref[...] = sel80
    xro_ref[...] = xro


def _tc3_body(feat_ref, dg_ref, h2_ref, live80_ref, bl_ref, Wr_ref, p_ref,
              x1_ref, x2_ref, W1_ref, b1_ref, W2_ref, b2_ref, W3_ref, b3_ref,
              cat_ref, prob_ref):
    h2 = h2_ref[...]
    out = _conv_out(feat_ref, dg_ref, h2, None, bl_ref, Wr_ref)
    _, _, _, x3 = _pool(out, live80_ref[...], p_ref, K3)
    z = jnp.concatenate([x1_ref[...], x2_ref[...], x3], axis=1)
    hc = jnp.maximum(jnp.dot(z, W1_ref[...], precision=_HP) + b1_ref[...], 0.0)
    hc = jnp.maximum(jnp.dot(hc, W2_ref[...], precision=_HP) + b2_ref[...], 0.0)
    prob = jnp.dot(hc, W3_ref[...], precision=_HP) + b3_ref[...]
    cat_ref[...] = jnp.concatenate([z, prob], axis=1)
    prob_ref[...] = prob


def _f32(shape):
    return jax.ShapeDtypeStruct(shape, jnp.float32)


def kernel(x, edge_index, batch, bn_gamma, bn_beta, c1_Wl, c1_bl, c1_Wr, p1,
           c2_Wl, c2_bl, c2_Wr, p2, c3_Wl, c3_bl, c3_Wr, p3,
           W1, b1, W2, b2, W3, b3):
    xp = jnp.pad(x, ((0, NPAD - N), (0, 0)))
    epad = jnp.full((2, EPAD - E), N, jnp.int32)
    edges_flat = jnp.concatenate([edge_index.astype(jnp.int32), epad],
                                 axis=1).reshape(2 * NWORK, NBLK, BLK)
    zeros_p = jnp.zeros((NPAD, WP), jnp.float32)
    zeros_q = jnp.zeros((NPAD, WQ), jnp.float32)
    rmask_col = jnp.where(jnp.arange(NPAD) < N, 1.0, 0.0).reshape(NPAD, 1)
    pad15 = jnp.zeros((NPAD, 15), jnp.float32)

    h0 = pl.pallas_call(_tc0_body, out_shape=_f32((NPAD, C)))(
        xp, bn_gamma.reshape(1, C), bn_beta.reshape(1, C))

    def agg_full(h, mcol):
        pa = _sc_aggregate(h[:, 0:WP], edges_flat, zeros_p, WP)
        tb = jnp.concatenate([h[:, WP:C], mcol, pad15], axis=1)
        pb = _sc_aggregate(tb, edges_flat, zeros_q, WQ)
        feat = jnp.concatenate([pa, pb[:, 0:C - WP]], axis=1)
        dg = pb[:, C - WP].reshape(2 * NROW, 128)
        return feat, dg

    feat1, dg1 = agg_full(h0, rmask_col)
    hnew1, sel1, x1 = pl.pallas_call(
        _tc1_body,
        out_shape=(_f32((NPAD, C)), _f32((NROW, 128)), _f32((1, 2 * C))),
    )(feat1, dg1, h0, c1_Wl, c1_bl.reshape(1, C), c1_Wr, p1.reshape(1, C))

    feat2, dg2 = agg_full(hnew1, sel1.reshape(NPAD, 1))
    y2, h2, sel2, x2 = pl.pallas_call(
        _tc2_body,
        out_shape=(_f32((NPAD, 64)), _f32((NPAD, C)), _f32((NROW, 128)),
                   _f32((1, 2 * C))),
    )(feat2, dg2, hnew1, sel1, c2_Wl, c2_bl.reshape(1, C), c2_Wr,
      p2.reshape(1, C), c3_Wl)

    ytable2 = jnp.concatenate([y2, sel2.reshape(NPAD, 1), pad15], axis=1)
    parts3 = _sc_aggregate(ytable2, edges_flat, zeros_p, WB)
    feat3 = parts3[:, 0:64]
    dg3 = parts3[:, 64].reshape(2 * NROW, 128)
    cat, prob = pl.pallas_call(
        _tc3_body,
        out_shape=(_f32((1, 650)), _f32((1, 10))),
    )(feat3, dg3, h2, sel2, c3_bl.reshape(1, 64), c3_Wr, p3.reshape(1, 64),
      x1, x2, W1, b1.reshape(1, 256), W2, b2.reshape(1, 128), W3,
      b3.reshape(1, 10))
    return cat, prob


# speedup vs baseline: 2.8790x; 2.8790x over previous
import functools
import math

import jax
import jax.numpy as jnp
from jax import lax
from jax.experimental import pallas as pl
from jax.experimental.pallas import tpu as pltpu
from jax.experimental.pallas import tpu_sc as plsc

N = 10000
E = 320000
C = 128

NPAD = 10240
NROW = NPAD // 128
NCORE, NSUB = 2, 16
NWORK = NCORE * NSUB
BLK = 128
NBLK = 80
EPT = NBLK * BLK
EPAD = NWORK * EPT
WP = 80
WQ = 64
WB = 80

K1 = 5000
K2 = 2500
K3 = 1250

_HP = lax.Precision.HIGHEST


def _sc_agg_body(width, table_hbm, edges_hbm, zeros_hbm, out_hbm,
                 acc, src_all, dst_all, r0b, r1b, r2b, r3b,
                 g0, g1, g2, g3, s0, s1, s2, s3):
    cid = lax.axis_index("c")
    sid = lax.axis_index("s")
    wid = cid * NSUB + sid
    rpt = NPAD // NSUB
    row0 = sid * rpt
    rows = (r0b, r1b, r2b, r3b)
    gsem = (g0, g1, g2, g3)
    ssem = (s0, s1, s2, s3)
    pltpu.sync_copy(zeros_hbm.at[pl.ds(row0, rpt)], acc.at[pl.ds(row0, rpt)])
    pltpu.sync_copy(edges_hbm.at[wid], src_all)
    pltpu.sync_copy(edges_hbm.at[NWORK + wid], dst_all)
    plsc.subcore_barrier()

    for k in range(3):
        pltpu.async_copy(table_hbm.at[src_all.at[k]], rows[k], gsem[k])

    def phase(i, k):
        kp = (k + 3) % 4

        @pl.when(jnp.logical_and(i >= 1, i <= NBLK - 4))
        def _():
            pltpu.make_async_copy(rows[kp], acc.at[dst_all.at[i - 1]],
                                  ssem[kp]).wait()

        @pl.when(i <= NBLK - 4)
        def _():
            pltpu.async_copy(table_hbm.at[src_all.at[i + 3]], rows[kp],
                             gsem[kp])

        pltpu.make_async_copy(table_hbm.at[src_all.at[i]], rows[k],
                              gsem[k]).wait()
        pltpu.async_copy(rows[k], acc.at[dst_all.at[i]], ssem[k], add=True)

    def group(j, carry):
        for k in range(4):
            phase(4 * j + k, k)
        return carry

    lax.fori_loop(0, NBLK // 4, group, 0)
    for k in range(4):
        pltpu.make_async_copy(rows[k], acc.at[dst_all.at[NBLK - 4 + k]],
                              ssem[k]).wait()
    plsc.subcore_barrier()
    pltpu.sync_copy(acc.at[pl.ds(row0, rpt)],
                    out_hbm.at[pl.ds(cid * NPAD + row0, rpt)])


def _sc_aggregate(table, edges3, zeros, width):
    mesh = plsc.VectorSubcoreMesh(core_axis_name="c", subcore_axis_name="s")
    fn = pl.kernel(
        functools.partial(_sc_agg_body, width),
        out_type=jax.ShapeDtypeStruct((2 * NPAD, width), jnp.float32),
        mesh=mesh,
        scratch_types=[
            pltpu.VMEM_SHARED((NPAD, width), jnp.float32),
            pltpu.VMEM((NBLK, BLK), jnp.int32),
            pltpu.VMEM((NBLK, BLK), jnp.int32),
        ] + [pltpu.VMEM((BLK, width), jnp.float32)] * 4
          + [pltpu.SemaphoreType.DMA] * 8,
        compiler_params=pltpu.CompilerParams(use_tc_tiling_on_sc=False),
    )
    return fn(table, edges3, zeros)


def _select_topk(score80, live80, kk):
    bits = lax.bitcast_convert_type(score80, jnp.int32)
    key = bits ^ ((bits >> 31) & jnp.int32(0x7FFFFFFF))
    ku = lax.bitcast_convert_type(key, jnp.uint32) ^ jnp.uint32(0x80000000)
    ku = jnp.where(live80 > 0, ku, jnp.uint32(0))

    def rbody(t, prefix):
        cand = prefix | (jnp.uint32(0x80000000) >> t.astype(jnp.uint32))
        cnt = jnp.sum(jnp.where(ku >= cand, 1, 0))
        return jnp.where(cnt >= kk, cand, prefix)

    t = lax.fori_loop(0, 32, rbody, jnp.uint32(0), unroll=True)
    c1 = jnp.sum(jnp.where(ku > t, 1, 0))
    r = kk - c1
    idx = (lax.broadcasted_iota(jnp.int32, (NROW, 128), 0) * 128
           + lax.broadcasted_iota(jnp.int32, (NROW, 128), 1))
    eq = ku == t

    def bbody(_, lohi):
        lo, hi = lohi
        mid = (lo + hi) // 2
        cnt = jnp.sum(jnp.where(eq & (idx <= mid), 1, 0))
        good = cnt >= r
        return (jnp.where(good, lo, mid + 1), jnp.where(good, mid, hi))

    lo, _ = lax.fori_loop(0, 14, bbody,
                          (jnp.int32(0), jnp.int32(NPAD - 1)), unroll=True)
    sel = (ku > t) | (eq & (idx <= lo))
    return jnp.where((live80 > 0) & sel, 1.0, 0.0)


def _conv_out(feat_ref, dg_ref, h, Wl_ref, bl_ref, Wr_ref):
    agg = feat_ref[0:NPAD] + feat_ref[NPAD:2 * NPAD]
    deg = _grid_to_col(dg_ref[0:NROW] + dg_ref[NROW:2 * NROW])
    mean = agg / jnp.maximum(deg, 1.0)
    pre = (bl_ref[...] + mean
           + jnp.dot(h, Wr_ref[...], precision=_HP)) if Wl_ref is None else (
        jnp.dot(mean, Wl_ref[...], precision=_HP) + bl_ref[...]
        + jnp.dot(h, Wr_ref[...], precision=_HP))
    return jnp.maximum(pre, 0.0)


def _col_to_grid(col):
    return jnp.concatenate(
        [col[i * 128:(i + 1) * 128, :].T for i in range(NROW)], axis=0)


def _grid_to_col(g):
    return jnp.concatenate([g[i:i + 1, :].T for i in range(NROW)], axis=0)


def _pool(out, live80, p_ref, kk):
    p = p_ref[...]
    pnorm = jnp.sqrt(jnp.sum(p * p))
    sc_col = jnp.tanh(jnp.sum(out * p, axis=1, keepdims=True)
                      / (pnorm + 1e-16))
    score80 = _col_to_grid(sc_col)
    sel80 = _select_topk(score80, live80, kk)
    selc = _grid_to_col(sel80)
    hnew = out * (sc_col * selc)
    mx = jnp.max(jnp.where(selc > 0.0, hnew, -jnp.inf), axis=0, keepdims=True)
    mn = jnp.sum(hnew, axis=0, keepdims=True) * (1.0 / kk)
    return hnew, selc, sel80, jnp.concatenate([mx, mn], axis=1)


def _tc0_body(x_ref, g_ref, b_ref, h_ref):
    x = x_ref[...]
    mu = jnp.sum(x, axis=0, keepdims=True) * (1.0 / N)
    d = x - mu
    var = (jnp.sum(d * d, axis=0, keepdims=True)
           - (NPAD - N) * mu * mu) * (1.0 / N)
    h = d / jnp.sqrt(var + 1e-5) * g_ref[...] + b_ref[...]
    rmask = jnp.where(
        lax.broadcasted_iota(jnp.int32, (NPAD, 1), 0) < N, 1.0, 0.0)
    h_ref[...] = h * rmask


def _tc1_body(feat_ref, dg_ref, h0_ref, Wl_ref, bl_ref, Wr_ref, p_ref,
              hnew_ref, sel80_ref, xro_ref):
    out = _conv_out(feat_ref, dg_ref, h0_ref[...], Wl_ref, bl_ref, Wr_ref)
    idx = (lax.broadcasted_iota(jnp.int32, (NROW, 128), 0) * 128
           + lax.broadcasted_iota(jnp.int32, (NROW, 128), 1))
    live80 = jnp.where(idx < N, 1.0, 0.0)
    hnew, selc, sel80, xro = _pool(out, live80, p_ref, K1)
    hnew_ref[...] = hnew
    sel80_ref[...] = sel80
    xro_ref[...] = xro


def _tc2_body(feat_ref, dg_ref, h1_ref, live80_ref, Wl_ref, bl_ref, Wr_ref,
              p_ref, Wlnext_ref, y_ref, h2_ref, sel80_ref, xro_ref):
    out = _conv_out(feat_ref, dg_ref, h1_ref[...], Wl_ref, bl_ref, Wr_ref)
    hnew, selc, sel80, xro = _pool(out, live80_ref[...], p_ref, K2)
    y_ref[...] = jnp.dot(hnew, Wlnext_ref[...], precision=_HP)
    h2_ref[...] = hnew
    sel80_ref[...] = sel80
    xro_ref[...] = xro


def _tc3_body(feat_ref, dg_ref, h2_ref, live80_ref, bl_ref, Wr_ref, p_ref,
              x1_ref, x2_ref, W1_ref, b1_ref, W2_ref, b2_ref, W3_ref, b3_ref,
              cat_ref, prob_ref):
    h2 = h2_ref[...]
    out = _conv_out(feat_ref, dg_ref, h2, None, bl_ref, Wr_ref)
    _, _, _, x3 = _pool(out, live80_ref[...], p_ref, K3)
    z = jnp.concatenate([x1_ref[...], x2_ref[...], x3], axis=1)
    hc = jnp.maximum(jnp.dot(z, W1_ref[...], precision=_HP) + b1_ref[...], 0.0)
    hc = jnp.maximum(jnp.dot(hc, W2_ref[...], precision=_HP) + b2_ref[...], 0.0)
    prob = jnp.dot(hc, W3_ref[...], precision=_HP) + b3_ref[...]
    cat_ref[...] = jnp.concatenate([z, prob], axis=1)
    prob_ref[...] = prob


def _f32(shape):
    return jax.ShapeDtypeStruct(shape, jnp.float32)


def kernel(x, edge_index, batch, bn_gamma, bn_beta, c1_Wl, c1_bl, c1_Wr, p1,
           c2_Wl, c2_bl, c2_Wr, p2, c3_Wl, c3_bl, c3_Wr, p3,
           W1, b1, W2, b2, W3, b3):
    xp = jnp.pad(x, ((0, NPAD - N), (0, 0)))
    pad_idx = N + (jnp.arange(EPAD - E, dtype=jnp.int32) % (NPAD - N))
    epad = jnp.stack([pad_idx, pad_idx])
    edges_flat = jnp.concatenate([edge_index.astype(jnp.int32), epad],
                                 axis=1).reshape(2 * NWORK, NBLK, BLK)
    zeros_p = jnp.zeros((NPAD, WP), jnp.float32)
    zeros_q = jnp.zeros((NPAD, WQ), jnp.float32)
    rmask_col = jnp.where(jnp.arange(NPAD) < N, 1.0, 0.0).reshape(NPAD, 1)
    pad15 = jnp.zeros((NPAD, 15), jnp.float32)

    h0 = pl.pallas_call(_tc0_body, out_shape=_f32((NPAD, C)))(
        xp, bn_gamma.reshape(1, C), bn_beta.reshape(1, C))

    def agg_full(h, mcol):
        pa = _sc_aggregate(h[:, 0:WP], edges_flat, zeros_p, WP)
        tb = jnp.concatenate([h[:, WP:C], mcol, pad15], axis=1)
        pb = _sc_aggregate(tb, edges_flat, zeros_q, WQ)
        feat = jnp.concatenate([pa, pb[:, 0:C - WP]], axis=1)
        dg = pb[:, C - WP].reshape(2 * NROW, 128)
        return feat, dg

    feat1, dg1 = agg_full(h0, rmask_col)
    hnew1, sel1, x1 = pl.pallas_call(
        _tc1_body,
        out_shape=(_f32((NPAD, C)), _f32((NROW, 128)), _f32((1, 2 * C))),
    )(feat1, dg1, h0, c1_Wl, c1_bl.reshape(1, C), c1_Wr, p1.reshape(1, C))

    feat2, dg2 = agg_full(hnew1, sel1.reshape(NPAD, 1))
    y2, h2, sel2, x2 = pl.pallas_call(
        _tc2_body,
        out_shape=(_f32((NPAD, 64)), _f32((NPAD, C)), _f32((NROW, 128)),
                   _f32((1, 2 * C))),
    )(feat2, dg2, hnew1, sel1, c2_Wl, c2_bl.reshape(1, C), c2_Wr,
      p2.reshape(1, C), c3_Wl)

    ytable2 = jnp.concatenate([y2, sel2.reshape(NPAD, 1), pad15], axis=1)
    parts3 = _sc_aggregate(ytable2, edges_flat, zeros_p, WB)
    feat3 = parts3[:, 0:64]
    dg3 = parts3[:, 64].reshape(2 * NROW, 128)
    cat, prob = pl.pallas_call(
        _tc3_body,
        out_shape=(_f32((1, 650)), _f32((1, 10))),
    )(feat3, dg3, h2, sel2, c3_bl.reshape(1, 64), c3_Wr, p3.reshape(1, 64),
      x1, x2, W1, b1.reshape(1, 256), W2, b2.reshape(1, 128), W3,
      b3.reshape(1, 10))
    return cat, prob


# speedup vs baseline: 2.8796x; 1.0002x over previous
import functools

import jax
import jax.numpy as jnp
from jax import lax
from jax.experimental import pallas as pl
from jax.experimental.pallas import tpu as pltpu
from jax.experimental.pallas import tpu_sc as plsc

N = 10000
E = 320000
C = 128

NPAD = 10240
NROW = NPAD // 128
NCORE, NSUB = 2, 16
NWORK = NCORE * NSUB
BLK = 128
NBLK = 80
EPT = NBLK * BLK
EPAD = NWORK * EPT
WP = 80
WQ = 64
WB = 80

K1 = 5000
K2 = 2500
K3 = 1250

_HP = lax.Precision.HIGHEST


def _sc_agg_body(width, table_hbm, edges_hbm, zeros_hbm, out_hbm,
                 acc, src_all, dst_all, r0b, r1b, r2b, r3b,
                 g0, g1, g2, g3, s0, s1, s2, s3):
    cid = lax.axis_index("c")
    sid = lax.axis_index("s")
    wid = cid * NSUB + sid
    rpt = NPAD // NSUB
    row0 = sid * rpt
    rows = (r0b, r1b, r2b, r3b)
    gsem = (g0, g1, g2, g3)
    ssem = (s0, s1, s2, s3)
    pltpu.sync_copy(zeros_hbm.at[pl.ds(row0, rpt)], acc.at[pl.ds(row0, rpt)])
    pltpu.sync_copy(edges_hbm.at[wid], src_all)
    pltpu.sync_copy(edges_hbm.at[NWORK + wid], dst_all)
    plsc.subcore_barrier()

    for k in range(3):
        pltpu.async_copy(table_hbm.at[src_all.at[k]], rows[k], gsem[k])

    def phase(i, k):
        kp = (k + 3) % 4

        @pl.when(jnp.logical_and(i >= 1, i <= NBLK - 4))
        def _():
            pltpu.make_async_copy(rows[kp], acc.at[dst_all.at[i - 1]],
                                  ssem[kp]).wait()

        @pl.when(i <= NBLK - 4)
        def _():
            pltpu.async_copy(table_hbm.at[src_all.at[i + 3]], rows[kp],
                             gsem[kp])

        pltpu.make_async_copy(table_hbm.at[src_all.at[i]], rows[k],
                              gsem[k]).wait()
        pltpu.async_copy(rows[k], acc.at[dst_all.at[i]], ssem[k], add=True)

    def group(j, carry):
        for k in range(4):
            phase(4 * j + k, k)
        return carry

    lax.fori_loop(0, NBLK // 4, group, 0)
    for k in range(4):
        pltpu.make_async_copy(rows[k], acc.at[dst_all.at[NBLK - 4 + k]],
                              ssem[k]).wait()
    plsc.subcore_barrier()
    pltpu.sync_copy(acc.at[pl.ds(row0, rpt)],
                    out_hbm.at[pl.ds(cid * NPAD + row0, rpt)])


def _sc_aggregate(table, edges3, zeros, width):
    mesh = plsc.VectorSubcoreMesh(core_axis_name="c", subcore_axis_name="s")
    fn = pl.kernel(
        functools.partial(_sc_agg_body, width),
        out_type=jax.ShapeDtypeStruct((2 * NPAD, width), jnp.float32),
        mesh=mesh,
        scratch_types=[
            pltpu.VMEM_SHARED((NPAD, width), jnp.float32),
            pltpu.VMEM((NBLK, BLK), jnp.int32),
            pltpu.VMEM((NBLK, BLK), jnp.int32),
        ] + [pltpu.VMEM((BLK, width), jnp.float32)] * 4
          + [pltpu.SemaphoreType.DMA] * 8,
        compiler_params=pltpu.CompilerParams(use_tc_tiling_on_sc=False),
    )
    return fn(table, edges3, zeros)


def _select_topk(score80, live80, kk):
    bits = lax.bitcast_convert_type(score80, jnp.int32)
    key = bits ^ ((bits >> 31) & jnp.int32(0x7FFFFFFF))
    ku = lax.bitcast_convert_type(key, jnp.uint32) ^ jnp.uint32(0x80000000)
    ku = jnp.where(live80 > 0, ku, jnp.uint32(0))

    def rbody(t, prefix):
        cand = prefix | (jnp.uint32(0x80000000) >> t.astype(jnp.uint32))
        cnt = jnp.sum(jnp.where(ku >= cand, 1, 0))
        return jnp.where(cnt >= kk, cand, prefix)

    t = lax.fori_loop(0, 32, rbody, jnp.uint32(0), unroll=True)
    c1 = jnp.sum(jnp.where(ku > t, 1, 0))
    r = kk - c1
    idx = (lax.broadcasted_iota(jnp.int32, (NROW, 128), 0) * 128
           + lax.broadcasted_iota(jnp.int32, (NROW, 128), 1))
    eq = ku == t

    def bbody(_, lohi):
        lo, hi = lohi
        mid = (lo + hi) // 2
        cnt = jnp.sum(jnp.where(eq & (idx <= mid), 1, 0))
        good = cnt >= r
        return (jnp.where(good, lo, mid + 1), jnp.where(good, mid, hi))

    lo, _ = lax.fori_loop(0, 14, bbody,
                          (jnp.int32(0), jnp.int32(NPAD - 1)), unroll=True)
    sel = (ku > t) | (eq & (idx <= lo))
    return jnp.where((live80 > 0) & sel, 1.0, 0.0)


def _conv_out(feat_ref, dg_ref, h, Wl_ref, bl_ref, Wr_ref):
    agg = feat_ref[0:NPAD] + feat_ref[NPAD:2 * NPAD]
    deg = _grid_to_col(dg_ref[0:NROW] + dg_ref[NROW:2 * NROW])
    mean = agg / jnp.maximum(deg, 1.0)
    pre = (bl_ref[...] + mean
           + jnp.dot(h, Wr_ref[...], precision=_HP)) if Wl_ref is None else (
        jnp.dot(mean, Wl_ref[...], precision=_HP) + bl_ref[...]
        + jnp.dot(h, Wr_ref[...], precision=_HP))
    return jnp.maximum(pre, 0.0)


def _col_to_grid(col):
    return jnp.concatenate(
        [col[i * 128:(i + 1) * 128, :].T for i in range(NROW)], axis=0)


def _grid_to_col(g):
    return jnp.concatenate([g[i:i + 1, :].T for i in range(NROW)], axis=0)


def _pool(out, live80, p_ref, kk):
    p = p_ref[...]
    pnorm = jnp.sqrt(jnp.sum(p * p))
    sc_col = jnp.tanh(jnp.sum(out * p, axis=1, keepdims=True)
                      / (pnorm + 1e-16))
    score80 = _col_to_grid(sc_col)
    sel80 = _select_topk(score80, live80, kk)
    selc = _grid_to_col(sel80)
    hnew = out * (sc_col * selc)
    mx = jnp.max(jnp.where(selc > 0.0, hnew, -jnp.inf), axis=0, keepdims=True)
    mn = jnp.sum(hnew, axis=0, keepdims=True) * (1.0 / kk)
    return hnew, selc, sel80, jnp.concatenate([mx, mn], axis=1)


def _tc0_body(x_ref, g_ref, b_ref, h_ref):
    x = x_ref[...]
    mu = jnp.sum(x, axis=0, keepdims=True) * (1.0 / N)
    d = x - mu
    var = (jnp.sum(d * d, axis=0, keepdims=True)
           - (NPAD - N) * mu * mu) * (1.0 / N)
    h = d / jnp.sqrt(var + 1e-5) * g_ref[...] + b_ref[...]
    rmask = jnp.where(
        lax.broadcasted_iota(jnp.int32, (NPAD, 1), 0) < N, 1.0, 0.0)
    h_ref[...] = h * rmask


def _tc1_body(feat_ref, dg_ref, h0_ref, Wl_ref, bl_ref, Wr_ref, p_ref,
              hnew_ref, sel80_ref, xro_ref):
    out = _conv_out(feat_ref, dg_ref, h0_ref[...], Wl_ref, bl_ref, Wr_ref)
    idx = (lax.broadcasted_iota(jnp.int32, (NROW, 128), 0) * 128
           + lax.broadcasted_iota(jnp.int32, (NROW, 128), 1))
    live80 = jnp.where(idx < N, 1.0, 0.0)
    hnew, selc, sel80, xro = _pool(out, live80, p_ref, K1)
    hnew_ref[...] = hnew
    sel80_ref[...] = sel80
    xro_ref[...] = xro


def _tc2_body(feat_ref, dg_ref, h1_ref, live80_ref, Wl_ref, bl_ref, Wr_ref,
              p_ref, Wlnext_ref, y_ref, h2_ref, sel80_ref, xro_ref):
    out = _conv_out(feat_ref, dg_ref, h1_ref[...], Wl_ref, bl_ref, Wr_ref)
    hnew, selc, sel80, xro = _pool(out, live80_ref[...], p_ref, K2)
    y_ref[...] = jnp.dot(hnew, Wlnext_ref[...], precision=_HP)
    h2_ref[...] = hnew
    sel80_ref[...] = sel80
    xro_ref[...] = xro


def _tc3_body(feat_ref, dg_ref, h2_ref, live80_ref, bl_ref, Wr_ref, p_ref,
              x1_ref, x2_ref, W1_ref, b1_ref, W2_ref, b2_ref, W3_ref, b3_ref,
              cat_ref, prob_ref):
    h2 = h2_ref[...]
    out = _conv_out(feat_ref, dg_ref, h2, None, bl_ref, Wr_ref)
    _, _, _, x3 = _pool(out, live80_ref[...], p_ref, K3)
    z = jnp.concatenate([x1_ref[...], x2_ref[...], x3], axis=1)
    hc = jnp.maximum(jnp.dot(z, W1_ref[...], precision=_HP) + b1_ref[...], 0.0)
    hc = jnp.maximum(jnp.dot(hc, W2_ref[...], precision=_HP) + b2_ref[...], 0.0)
    prob = jnp.dot(hc, W3_ref[...], precision=_HP) + b3_ref[...]
    cat_ref[...] = jnp.concatenate([z, prob], axis=1)
    prob_ref[...] = prob


def _f32(shape):
    return jax.ShapeDtypeStruct(shape, jnp.float32)


def kernel(x, edge_index, batch, bn_gamma, bn_beta, c1_Wl, c1_bl, c1_Wr, p1,
           c2_Wl, c2_bl, c2_Wr, p2, c3_Wl, c3_bl, c3_Wr, p3,
           W1, b1, W2, b2, W3, b3):
    xp = jnp.pad(x, ((0, NPAD - N), (0, 0)))
    pad_idx = N + (jnp.arange(EPAD - E, dtype=jnp.int32) % (NPAD - N))
    epad = jnp.stack([pad_idx, pad_idx])
    edges_flat = jnp.concatenate([edge_index.astype(jnp.int32), epad],
                                 axis=1).reshape(2 * NWORK, NBLK, BLK)
    zeros_p = jnp.zeros((NPAD, WP), jnp.float32)
    zeros_q = jnp.zeros((NPAD, WQ), jnp.float32)
    rmask_col = jnp.where(jnp.arange(NPAD) < N, 1.0, 0.0).reshape(NPAD, 1)
    pad15 = jnp.zeros((NPAD, 15), jnp.float32)

    h0 = pl.pallas_call(_tc0_body, out_shape=_f32((NPAD, C)))(
        xp, bn_gamma.reshape(1, C), bn_beta.reshape(1, C))

    def agg_full(h, mcol):
        pa = _sc_aggregate(h[:, 0:WP], edges_flat, zeros_p, WP)
        tb = jnp.concatenate([h[:, WP:C], mcol, pad15], axis=1)
        pb = _sc_aggregate(tb, edges_flat, zeros_q, WQ)
        feat = jnp.concatenate([pa, pb[:, 0:C - WP]], axis=1)
        dg = pb[:, C - WP].reshape(2 * NROW, 128)
        return feat, dg

    feat1, dg1 = agg_full(h0, rmask_col)
    hnew1, sel1, x1 = pl.pallas_call(
        _tc1_body,
        out_shape=(_f32((NPAD, C)), _f32((NROW, 128)), _f32((1, 2 * C))),
    )(feat1, dg1, h0, c1_Wl, c1_bl.reshape(1, C), c1_Wr, p1.reshape(1, C))

    feat2, dg2 = agg_full(hnew1, sel1.reshape(NPAD, 1))
    y2, h2, sel2, x2 = pl.pallas_call(
        _tc2_body,
        out_shape=(_f32((NPAD, 64)), _f32((NPAD, C)), _f32((NROW, 128)),
                   _f32((1, 2 * C))),
    )(feat2, dg2, hnew1, sel1, c2_Wl, c2_bl.reshape(1, C), c2_Wr,
      p2.reshape(1, C), c3_Wl)

    ytable2 = jnp.concatenate([y2, sel2.reshape(NPAD, 1), pad15], axis=1)
    parts3 = _sc_aggregate(ytable2, edges_flat, zeros_p, WB)
    feat3 = parts3[:, 0:64]
    dg3 = parts3[:, 64].reshape(2 * NROW, 128)
    cat, prob = pl.pallas_call(
        _tc3_body,
        out_shape=(_f32((1, 650)), _f32((1, 10))),
    )(feat3, dg3, h2, sel2, c3_bl.reshape(1, 64), c3_Wr, p3.reshape(1, 64),
      x1, x2, W1, b1.reshape(1, 256), W2, b2.reshape(1, 128), W3,
      b3.reshape(1, 10))
    return cat, prob
